# Initial kernel scaffold; baseline (speedup 1.0000x reference)
import jax
import jax.numpy as jnp
from jax.experimental import pallas as pl


def kernel(a, t, v, edge_index, edge_type, batch, params):
    raise NotImplementedError("write your pallas kernel here")


# speedup vs baseline: 10.3639x; 10.3639x over previous
import jax
import jax.numpy as jnp
from jax import lax
from jax.experimental import pallas as pl
from jax.experimental.pallas import tpu as pltpu
from jax.experimental.pallas import tpu_sc as plsc

_N = 10000
_E = 160000
_R = 5
_B = 64
_H = 128
_NC = 2
_NS = 16
_NW = _NC * _NS
_CH = 128
_NCHUNK = _E // _CH
_DEGP = 51200

f32 = jnp.float32
i32 = jnp.int32


def _ln(x, g, b):
    m = jnp.mean(x, axis=-1, keepdims=True)
    va = jnp.mean((x - m) * (x - m), axis=-1, keepdims=True)
    return (x - m) * lax.rsqrt(va + 1e-5) * g + b


def _enc_body(a_ref, t_ref, v_ref,
              aW0, ab0, aW1, ab1, alg, alb,
              tW1, tb1, tlg, tlb, tW2, tb2,
              vW1, vb1, vlg, vlb, vW2, vb2,
              wcat_ref, x_ref, y_ref):
    def lstm_pair(xin, W, bvec):
        gates = jnp.dot(xin, W[...], preferred_element_type=f32) + bvec[...]
        gi = gates[:, 0:128]
        gg = gates[:, 128:256]
        go = gates[:, 256:384]
        return jax.nn.sigmoid(go) * jnp.tanh(jax.nn.sigmoid(gi) * jnp.tanh(gg))

    h0 = lstm_pair(a_ref[...], aW0, ab0)
    h1 = lstm_pair(h0, aW1, ab1)
    za = _ln(h1, alg[...], alb[...])

    th = jnp.dot(t_ref[...], tW1[...], preferred_element_type=f32) + tb1[...]
    th = jax.nn.gelu(_ln(th, tlg[...], tlb[...]))
    zt = jnp.dot(th, tW2[...], preferred_element_type=f32) + tb2[...]

    vh = jnp.dot(v_ref[...], vW1[...], preferred_element_type=f32) + vb1[...]
    vh = jax.nn.gelu(_ln(vh, vlg[...], vlb[...]))
    zv = jnp.dot(vh, vW2[...], preferred_element_type=f32) + vb2[...]

    x = (za + zt + zv) * (1.0 / 3.0)
    x_ref[...] = x
    for r in range(_R + 1):
        y_ref[r] = jnp.dot(x, wcat_ref[r], preferred_element_type=f32)


def _lstm_weights(p, lvl, in_dim):
    cols, bs = [], []
    for gi in (0, 2, 3):
        for d in ('f', 'r'):
            W = p['l%d%s_Wih' % (lvl, d)]
            b = p['l%d%s_bih' % (lvl, d)] + p['l%d%s_bhh' % (lvl, d)]
            cols.append(W[gi * 64:(gi + 1) * 64].T)
            bs.append(b[gi * 64:(gi + 1) * 64])
    return (jnp.concatenate(cols, axis=1),
            jnp.concatenate(bs).reshape(1, 384))


def _encode(a, t, v, params, Wcat1):
    au, tx, vi = params['audio'], params['text'], params['visual']
    aW0, ab0 = _lstm_weights(au, 0, 74)
    aW1, ab1 = _lstm_weights(au, 1, 128)
    r2 = lambda z: z.reshape(1, -1)
    BLK = 1000
    grid = (_N // BLK,)
    row = lambda i: (i, 0)
    full = lambda i: (0, 0)
    ins = [a, t, v,
           aW0, ab0, aW1, ab1, r2(au['ln_g']), r2(au['ln_b']),
           tx['W1'], r2(tx['b1']), r2(tx['ln_g']), r2(tx['ln_b']), tx['W2'], r2(tx['b2']),
           vi['W1'], r2(vi['b1']), r2(vi['ln_g']), r2(vi['ln_b']), vi['W2'], r2(vi['b2']),
           Wcat1]
    in_specs = [pl.BlockSpec((BLK, a.shape[1]), row),
                pl.BlockSpec((BLK, t.shape[1]), row),
                pl.BlockSpec((BLK, v.shape[1]), row)]
    in_specs += [pl.BlockSpec(w.shape, full) for w in ins[3:-1]]
    in_specs += [pl.BlockSpec(Wcat1.shape, lambda i: (0, 0, 0))]
    return pl.pallas_call(
        _enc_body,
        grid=grid,
        in_specs=in_specs,
        out_specs=[pl.BlockSpec((BLK, _H), row),
                   pl.BlockSpec((_R + 1, BLK, _H), lambda i: (0, i, 0))],
        out_shape=[jax.ShapeDtypeStruct((_N, _H), f32),
                   jax.ShapeDtypeStruct((_R + 1, _N, _H), f32)],
    )(*ins)


def _comb_rel_body(p_ref, yr_ref, b_ref, g_ref, lb_ref, xin_ref, w_ref,
                   h_ref, y2_ref):
    m = p_ref[0] + p_ref[1] + yr_ref[0] + b_ref[...]
    h = jax.nn.gelu(_ln(m, g_ref[...], lb_ref[...])) + xin_ref[...]
    h_ref[...] = h
    for r in range(_R + 1):
        y2_ref[r] = jnp.dot(h, w_ref[r], preferred_element_type=f32)


def _comb_rel(part, y, bias, lng, lnb, xin, Wcat2):
    BLK = 2000
    nb = _N // BLK
    return pl.pallas_call(
        _comb_rel_body,
        grid=(nb,),
        in_specs=[pl.BlockSpec((_NC, BLK, _H), lambda b: (0, b, 0)),
                  pl.BlockSpec((1, BLK, _H), lambda b: (_R, b, 0)),
                  pl.BlockSpec((1, _H), lambda b: (0, 0)),
                  pl.BlockSpec((1, _H), lambda b: (0, 0)),
                  pl.BlockSpec((1, _H), lambda b: (0, 0)),
                  pl.BlockSpec((BLK, _H), lambda b: (b, 0)),
                  pl.BlockSpec((_R + 1, _H, _H), lambda b: (0, 0, 0))],
        out_specs=[pl.BlockSpec((BLK, _H), lambda b: (b, 0)),
                   pl.BlockSpec((_R + 1, BLK, _H), lambda b: (0, b, 0))],
        out_shape=[jax.ShapeDtypeStruct((_N, _H), f32),
                   jax.ShapeDtypeStruct((_R + 1, _N, _H), f32)],
    )(part, y, bias.reshape(1, _H), lng.reshape(1, _H), lnb.reshape(1, _H),
      xin, Wcat2)


def _sc_mesh():
    return plsc.VectorSubcoreMesh(core_axis_name="c", subcore_axis_name="s",
                                  num_cores=_NC, num_subcores=_NS)


_DCH = 1000
_DGRP = -(-_DCH // 16)
_DBUF = _DGRP * 16


def _deg_body(dst_hbm, et_hbm, out_hbm, dstv, etv, deg_l):
    cid = lax.axis_index("c")
    sid = lax.axis_index("s")
    wid = sid * _NC + cid
    zero = jnp.zeros((16,), f32)

    def zb(i, _):
        deg_l[pl.ds(i * 16, 16)] = zero
        return 0
    lax.fori_loop(0, _DEGP // 16, zb, 0)

    ones = jnp.ones((16,), f32)
    tailmask = lax.iota(i32, 16) < (_DCH - (_DGRP - 1) * 16)

    def chunk(i, _):
        base = (i * _NW + wid) * _DCH
        pltpu.sync_copy(dst_hbm.at[pl.ds(base, _DCH)], dstv.at[pl.ds(0, _DCH)])
        pltpu.sync_copy(et_hbm.at[pl.ds(base, _DCH)], etv.at[pl.ds(0, _DCH)])
        for j in range(_DGRP):
            d = dstv[pl.ds(j * 16, 16)]
            e = etv[pl.ds(j * 16, 16)]
            if (j + 1) * 16 <= _DCH:
                plsc.addupdate_scatter(deg_l, [d * _R + e], ones)
            else:
                plsc.addupdate_scatter(deg_l, [d * _R + e], ones, mask=tailmask)
        return 0
    lax.fori_loop(0, _E // _DCH // _NW, chunk, 0)
    pltpu.sync_copy(deg_l, out_hbm.at[wid])


def _deg(dst, et):
    return pl.kernel(
        _deg_body,
        out_type=jax.ShapeDtypeStruct((_NW, _DEGP), f32),
        mesh=_sc_mesh(),
        compiler_params=pltpu.CompilerParams(needs_layout_passes=False),
        scratch_types=[pltpu.VMEM((_DBUF,), i32),
                       pltpu.VMEM((_DBUF,), i32),
                       pltpu.VMEM((_DEGP,), f32)],
    )(dst, et)


def _rnorm_body(degp_ref, rn_ref):
    d = jnp.sum(degp_ref[...], axis=0)
    rn_ref[...] = 1.0 / jnp.maximum(d, 1.0)


def _rnorm(degp):
    degp3 = degp.reshape(_NW, _DEGP // 128, 128)
    return pl.pallas_call(
        _rnorm_body,
        in_specs=[pl.BlockSpec(degp3.shape, lambda: (0, 0, 0))],
        out_specs=pl.BlockSpec((_DEGP // 128, 128), lambda: (0, 0)),
        out_shape=jax.ShapeDtypeStruct((_DEGP // 128, 128), f32),
    )(degp3)


_EP = 163840
_NCHP = _EP // _CH
_RUN = 1024
_NRUN = _EP // _RUN


def _prep_body(src_hbm, dst_hbm, et_hbm, rn_hbm, pk_hbm,
               srcv, dstv, etv, pk, rn_v):
    cid = lax.axis_index("c")
    sid = lax.axis_index("s")
    wid = sid * _NC + cid
    pltpu.sync_copy(rn_hbm, rn_v)

    def run(i, _):
        r = i * _NW + wid
        base = r * _RUN
        pltpu.sync_copy(src_hbm.at[pl.ds(base, _RUN)], srcv)
        pltpu.sync_copy(dst_hbm.at[pl.ds(base, _RUN)], dstv)
        pltpu.sync_copy(et_hbm.at[pl.ds(base, _RUN)], etv)
        for sub in range(_RUN // _CH):
            for g in range(_CH // 16):
                off = sub * _CH + g * 16
                s = srcv[pl.ds(off, 16)]
                d = dstv[pl.ds(off, 16)]
                e = etv[pl.ds(off, 16)]
                pk[sub, pl.ds(g * 16, 16)] = e * _N + s
                pk[sub, pl.ds(_CH + g * 16, 16)] = d
                pk[sub, pl.ds(2 * _CH + g * 16, 16)] = plsc.bitcast(
                    plsc.load_gather(rn_v, [d * _R + e]), i32)
        pltpu.sync_copy(pk, pk_hbm.at[pl.ds(r * (_RUN // _CH), _RUN // _CH)])
        return 0
    lax.fori_loop(0, _NRUN // _NW, run, 0)


def _prep(src, dst, et, rn_flat):
    return pl.kernel(
        _prep_body,
        out_type=jax.ShapeDtypeStruct((_NCHP, 3 * _CH), i32),
        mesh=_sc_mesh(),
        compiler_params=pltpu.CompilerParams(needs_layout_passes=False),
        scratch_types=[pltpu.VMEM((_RUN,), i32),
                       pltpu.VMEM((_RUN,), i32),
                       pltpu.VMEM((_RUN,), i32),
                       pltpu.VMEM((_RUN // _CH, 3 * _CH), i32),
                       pltpu.VMEM((_DEGP,), f32)],
    )(src, dst, et, rn_flat)


def _agg_body(y_hbm, pk_hbm, part_hbm,
              pk0, pk1, gid0, gid1, dstb0, dstb1,
              rows0, rows1, acc, sem0, sem1, ssem0, ssem1):
    cid = lax.axis_index("c")
    sid = lax.axis_index("s")
    wid = sid * _NC + cid
    pkb = (pk0, pk1)
    gid = (gid0, gid1)
    dstb = (dstb0, dstb1)
    rows = (rows0, rows1)
    sem = (sem0, sem1)
    ssem = (ssem0, ssem1)

    zero = jnp.zeros((16,), f32)

    def zb(k, _):
        for j in range(_H // 16):
            rows0[k, pl.ds(j * 16, 16)] = zero
        return 0
    lax.fori_loop(0, _CH, zb, 0)

    def for_each_span(do104, do128):
        @pl.when(sid < 15)
        def _():
            start = pl.multiple_of(sid * 624, 8)
            for c in range(6):
                do104(start + c * 104)

        @pl.when(sid == 15)
        def _():
            for c in range(5):
                do128(15 * 624 + c * 128)

    for_each_span(
        lambda off: pltpu.sync_copy(rows0.at[pl.ds(0, 104)],
                                    acc.at[pl.ds(off, 104)]),
        lambda off: pltpu.sync_copy(rows0, acc.at[pl.ds(off, 128)]))
    plsc.subcore_barrier()

    _NFULL = 39

    def load_and_issue(j, b, wait_scatter):
        pltpu.sync_copy(pk_hbm.at[j * _NW + wid], pkb[b])
        for g in range(_CH // 16):
            gid[b][pl.ds(g * 16, 16)] = pkb[b][pl.ds(g * 16, 16)]
            dstb[b][pl.ds(g * 16, 16)] = pkb[b][pl.ds(_CH + g * 16, 16)]
        pltpu.async_copy(y_hbm.at[gid[b]], rows[b], sem[b])

    def finish(b):
        pltpu.make_async_copy(y_hbm.at[gid[b]], rows[b], sem[b]).wait()

        def scale(gk, _):
            nv = plsc.bitcast(pkb[b][pl.ds(2 * _CH + gk * 16, 16)], f32)
            for l in range(16):
                sc = nv[l]
                r = gk * 16 + l
                for jj in range(_H // 16):
                    rows[b][r, pl.ds(jj * 16, 16)] = (
                        rows[b][r, pl.ds(jj * 16, 16)] * sc)
            return 0
        lax.fori_loop(0, _CH // 16, scale, 0)
        pltpu.sync_copy(rows[b], acc.at[dstb[b]], add=True)

    load_and_issue(0, 0, None)
    true_ = jnp.bool_(True)

    def pair(g, _):
        @pl.when(2 * g + 1 < _NFULL)
        def _():
            load_and_issue(2 * g + 1, 1, g >= 1)
        finish(0)

        @pl.when(2 * g + 2 < _NFULL)
        def _():
            load_and_issue(2 * g + 2, 0, true_)

        @pl.when(2 * g + 1 < _NFULL)
        def _():
            finish(1)
        return 0
    lax.fori_loop(0, (_NFULL + 1) // 2, pair, 0)

    @pl.when(wid < _NCHUNK - _NFULL * _NW)
    def _():
        load_and_issue(_NFULL, 0, true_)
        finish(0)
    plsc.subcore_barrier()
    for_each_span(
        lambda off: pltpu.sync_copy(acc.at[pl.ds(off, 104)],
                                    part_hbm.at[cid, pl.ds(off, 104)]),
        lambda off: pltpu.sync_copy(acc.at[pl.ds(off, 128)],
                                    part_hbm.at[cid, pl.ds(off, 128)]))


def _agg(y, packed):
    return pl.kernel(
        _agg_body,
        out_type=jax.ShapeDtypeStruct((_NC, _N, _H), f32),
        mesh=_sc_mesh(),
        compiler_params=pltpu.CompilerParams(needs_layout_passes=False),
        scratch_types=[pltpu.VMEM((3 * _CH,), i32),
                       pltpu.VMEM((3 * _CH,), i32),
                       pltpu.VMEM((_CH,), i32),
                       pltpu.VMEM((_CH,), i32),
                       pltpu.VMEM((_CH,), i32),
                       pltpu.VMEM((_CH,), i32),
                       pltpu.VMEM((_CH, _H), f32),
                       pltpu.VMEM((_CH, _H), f32),
                       pltpu.VMEM_SHARED((_N, _H), f32),
                       pltpu.SemaphoreType.DMA,
                       pltpu.SemaphoreType.DMA,
                       pltpu.SemaphoreType.DMA,
                       pltpu.SemaphoreType.DMA],
    )(y, packed)


def _comb_pool_body(p_ref, yr_ref, b_ref, g_ref, lb_ref, xin_ref, bat_ref,
                    wp_ref, bp_ref, out_ref, sum_acc, cnt_acc, max_acc):
    nb = pl.program_id(0)

    @pl.when(nb == 0)
    def _():
        sum_acc[...] = jnp.zeros_like(sum_acc)
        cnt_acc[...] = jnp.zeros_like(cnt_acc)
        max_acc[...] = jnp.full_like(max_acc, -1e30)

    m = p_ref[0] + p_ref[1] + yr_ref[0] + b_ref[...]
    h = jax.nn.gelu(_ln(m, g_ref[...], lb_ref[...])) + xin_ref[...]

    bcol = bat_ref[0, 0, :].reshape(-1, 1)
    onehot = (bcol == lax.broadcasted_iota(i32, (1, _B), 1)).astype(f32)
    sum_acc[...] += lax.dot_general(onehot, h, (((0,), (0,)), ((), ())),
                                    preferred_element_type=f32)
    cnt_acc[...] += lax.dot_general(onehot, jnp.ones_like(h),
                                    (((0,), (0,)), ((), ())),
                                    preferred_element_type=f32)
    mrows = [jnp.max(jnp.where(bcol == bb, h, -1e30), axis=0) for bb in range(_B)]
    max_acc[...] = jnp.maximum(max_acc[...], jnp.stack(mrows))

    @pl.when(nb == pl.num_programs(0) - 1)
    def _():
        cnt = cnt_acc[...]
        mean_p = sum_acc[...] / jnp.maximum(cnt, 1.0)
        max_p = jnp.where(cnt > 0, max_acc[...], 0.0)
        pooled = jnp.concatenate([mean_p, max_p], axis=-1)
        out_ref[...] = jnp.dot(pooled, wp_ref[...],
                               preferred_element_type=f32) + bp_ref[...]


def _comb_pool(part, y, bias, lng, lnb, xin, batch, Wpool, bpool):
    BLK = 2000
    nb = _N // BLK
    batch3 = batch.reshape(nb, 1, BLK)
    return pl.pallas_call(
        _comb_pool_body,
        grid=(nb,),
        in_specs=[pl.BlockSpec((_NC, BLK, _H), lambda b: (0, b, 0)),
                  pl.BlockSpec((1, BLK, _H), lambda b: (_R, b, 0)),
                  pl.BlockSpec((1, _H), lambda b: (0, 0)),
                  pl.BlockSpec((1, _H), lambda b: (0, 0)),
                  pl.BlockSpec((1, _H), lambda b: (0, 0)),
                  pl.BlockSpec((BLK, _H), lambda b: (b, 0)),
                  pl.BlockSpec((1, 1, BLK), lambda b: (b, 0, 0)),
                  pl.BlockSpec((2 * _H, _H), lambda b: (0, 0)),
                  pl.BlockSpec((1, _H), lambda b: (0, 0))],
        out_specs=pl.BlockSpec((_B, _H), lambda b: (0, 0)),
        out_shape=jax.ShapeDtypeStruct((_B, _H), f32),
        scratch_shapes=[pltpu.VMEM((_B, _H), f32),
                        pltpu.VMEM((_B, _H), f32),
                        pltpu.VMEM((_B, _H), f32)],
    )(part, y, bias.reshape(1, _H), lng.reshape(1, _H), lnb.reshape(1, _H),
      xin, batch3, Wpool, bpool.reshape(1, _H))


def kernel(a, t, v, edge_index, edge_type, batch, params):
    src = edge_index[0].astype(i32)
    dst = edge_index[1].astype(i32)
    et = edge_type.astype(i32)
    g = params['gnn']

    Wcat1 = jnp.concatenate([g['Wrel1'], g['Wroot1'][None]], axis=0)
    Wcat2 = jnp.concatenate([g['Wrel2'], g['Wroot2'][None]], axis=0)

    x, y1 = _encode(a, t, v, params, Wcat1)
    degp = _deg(dst, et)
    rn_flat = _rnorm(degp).reshape(_DEGP)
    pad = (0, _EP - _E)
    packed = _prep(jnp.pad(src, pad), jnp.pad(dst, pad), jnp.pad(et, pad),
                   rn_flat)

    part1 = _agg(y1.reshape((_R + 1) * _N, _H), packed)
    h, y2 = _comb_rel(part1, y1, g['b1'], g['ln1_g'], g['ln1_b'], x, Wcat2)
    part2 = _agg(y2.reshape((_R + 1) * _N, _H), packed)
    return _comb_pool(part2, y2, g['b2'], g['ln2_g'], g['ln2_b'], h,
                      batch, g['Wpool'], g['bpool'])
